# baseline (speedup 1.0000x reference)
import functools

import jax
import jax.numpy as jnp
from jax import lax
from jax.experimental import pallas as pl
from jax.experimental.pallas import tpu as pltpu
from jax.experimental.pallas import tpu_sc as plsc

EMBED = 32
TOKENS = 131072
BATCH = 8
FLAT = TOKENS * EMBED

NC = 2
NS = 16
NW = NC * NS
ELEMS_PER_W = FLAT // NW
CHUNK_ELEMS = 8192
NCHUNK = ELEMS_PER_W // CHUNK_ELEMS
NITER = NCHUNK * BATCH
NTOK_BUF = 10
PREF = 6

def _body(tok_hbm, pos_hbm, out_hbm, *refs):
    pos_v = refs[0:2]
    tok_v = refs[2:2 + NTOK_BUF]
    sem_pos = refs[2 + NTOK_BUF:4 + NTOK_BUF]
    sem_in = refs[4 + NTOK_BUF:4 + 2 * NTOK_BUF]
    sem_out = refs[4 + 2 * NTOK_BUF:4 + 3 * NTOK_BUF]

    wid = lax.axis_index("s") * NC + lax.axis_index("c")
    wbase = wid * ELEMS_PER_W

    def pos_in(j):
        return pltpu.async_copy(
            pos_hbm.at[pl.ds(wbase + j * CHUNK_ELEMS, CHUNK_ELEMS)],
            pos_v[j % 2], sem_pos[j % 2])

    def tok_in(it):
        j, b = divmod(it, BATCH)
        off = b * FLAT + wbase + j * CHUNK_ELEMS
        return pltpu.async_copy(
            tok_hbm.at[pl.ds(off, CHUNK_ELEMS)],
            tok_v[it % NTOK_BUF], sem_in[it % NTOK_BUF])

    def tok_out(it):
        j, b = divmod(it, BATCH)
        off = b * FLAT + wbase + j * CHUNK_ELEMS
        return pltpu.async_copy(
            tok_v[it % NTOK_BUF],
            out_hbm.at[pl.ds(off, CHUNK_ELEMS)],
            sem_out[it % NTOK_BUF])

    pending_pos = pos_in(0)
    pending_in = {i: tok_in(i) for i in range(PREF)}
    pending_out = {}

    for it in range(NITER):
        j, b = divmod(it, BATCH)
        if b == 0:
            pending_pos.wait()
            if j + 1 < NCHUNK:
                pending_pos = pos_in(j + 1)
        nx = it + PREF
        if nx < NITER:
            if nx - NTOK_BUF in pending_out:
                pending_out.pop(nx - NTOK_BUF).wait()
            pending_in[nx] = tok_in(nx)
        pending_in.pop(it).wait()

        tok_buf = tok_v[it % NTOK_BUF]
        pos_buf = pos_v[j % 2]

        @plsc.parallel_loop(0, CHUNK_ELEMS, step=16, unroll=8)
        def _(i):
            plsc.addupdate(tok_buf.at[pl.ds(i, 16)], pos_buf[pl.ds(i, 16)])

        pending_out[it] = tok_out(it)

    for it in sorted(pending_out):
        pending_out[it].wait()


@functools.cache
def _posenc_sc():
    mesh = plsc.VectorSubcoreMesh(core_axis_name="c", subcore_axis_name="s")
    return pl.kernel(
        _body,
        out_type=jax.ShapeDtypeStruct((BATCH * FLAT,), jnp.float32),
        mesh=mesh,
        scratch_types=(
            [pltpu.VMEM((CHUNK_ELEMS,), jnp.float32) for _ in range(2)]
            + [pltpu.VMEM((CHUNK_ELEMS,), jnp.float32) for _ in range(NTOK_BUF)]
            + [pltpu.SemaphoreType.DMA for _ in range(2 + 2 * NTOK_BUF)]
        ),
    )


def _to_physical_tok(x):
    return (x.transpose(0, 2, 1)
             .reshape(BATCH, EMBED // 8, 8, TOKENS // 128, 128)
             .transpose(0, 1, 3, 2, 4)
             .reshape(BATCH * FLAT))


def _from_physical_tok(x):
    return (x.reshape(BATCH, EMBED // 8, TOKENS // 128, 8, 128)
             .transpose(0, 1, 3, 2, 4)
             .reshape(BATCH, EMBED, TOKENS)
             .transpose(0, 2, 1))


def _to_physical_pos(x):
    return (x.transpose(1, 0)
             .reshape(EMBED // 8, 8, TOKENS // 128, 128)
             .transpose(0, 2, 1, 3)
             .reshape(FLAT))


def kernel(encoded_tokens, position_embedding):
    tok_lin = _to_physical_tok(encoded_tokens)
    pos_lin = _to_physical_pos(position_embedding)
    out_lin = _posenc_sc()(tok_lin, pos_lin)
    return _from_physical_tok(out_lin)


# speedup vs baseline: 1.0185x; 1.0185x over previous
import functools

import jax
import jax.numpy as jnp
from jax import lax
from jax.experimental import pallas as pl
from jax.experimental.pallas import tpu as pltpu
from jax.experimental.pallas import tpu_sc as plsc

EMBED = 32
TOKENS = 131072
BATCH = 8
FLAT = TOKENS * EMBED

NC = 2
NS = 16
NW = NC * NS
ELEMS_PER_W = FLAT // NW
CHUNK_ELEMS = 16384
HALF = CHUNK_ELEMS // 2
NCHUNK = ELEMS_PER_W // CHUNK_ELEMS
NITER = NCHUNK * BATCH
NTOK_BUF = 4
PREF = 2


def _body(tok_hbm, pos_hbm, out_hbm, *refs):
    pos_v = refs[0:2]
    tok_v = refs[2:2 + NTOK_BUF]
    sem_pos = refs[2 + NTOK_BUF:4 + NTOK_BUF]
    sem_in = refs[4 + NTOK_BUF:4 + 3 * NTOK_BUF]
    sem_out = refs[4 + 3 * NTOK_BUF:4 + 5 * NTOK_BUF]

    wid = lax.axis_index("s") * NC + lax.axis_index("c")
    wbase = wid * ELEMS_PER_W

    def pos_in(j):
        return [pltpu.async_copy(
            pos_hbm.at[pl.ds(wbase + j * CHUNK_ELEMS, CHUNK_ELEMS)],
            pos_v[j % 2], sem_pos[j % 2])]

    def tok_in(it):
        j, b = divmod(it, BATCH)
        off = b * FLAT + wbase + j * CHUNK_ELEMS
        p = it % NTOK_BUF
        return [
            pltpu.async_copy(
                tok_hbm.at[pl.ds(off + h * HALF, HALF)],
                tok_v[p].at[pl.ds(h * HALF, HALF)],
                sem_in[2 * p + h])
            for h in range(2)
        ]

    def tok_out(it):
        j, b = divmod(it, BATCH)
        off = b * FLAT + wbase + j * CHUNK_ELEMS
        p = it % NTOK_BUF
        return [
            pltpu.async_copy(
                tok_v[p].at[pl.ds(h * HALF, HALF)],
                out_hbm.at[pl.ds(off + h * HALF, HALF)],
                sem_out[2 * p + h])
            for h in range(2)
        ]

    def wait(descs):
        for d in descs:
            d.wait()

    pending_pos = pos_in(0)
    pending_in = {i: tok_in(i) for i in range(PREF)}
    pending_out = {}

    for it in range(NITER):
        j, b = divmod(it, BATCH)
        if b == 0:
            wait(pending_pos)
            if j + 1 < NCHUNK:
                pending_pos = pos_in(j + 1)
        nx = it + PREF
        if nx < NITER:
            if nx - NTOK_BUF in pending_out:
                wait(pending_out.pop(nx - NTOK_BUF))
            pending_in[nx] = tok_in(nx)
        wait(pending_in.pop(it))

        tok_buf = tok_v[it % NTOK_BUF]
        pos_buf = pos_v[j % 2]

        @plsc.parallel_loop(0, CHUNK_ELEMS, step=16, unroll=8)
        def _(i):
            plsc.addupdate(tok_buf.at[pl.ds(i, 16)], pos_buf[pl.ds(i, 16)])

        pending_out[it] = tok_out(it)

    for it in sorted(pending_out):
        wait(pending_out[it])


@functools.cache
def _posenc_sc():
    mesh = plsc.VectorSubcoreMesh(core_axis_name="c", subcore_axis_name="s")
    return pl.kernel(
        _body,
        out_type=jax.ShapeDtypeStruct((BATCH * FLAT,), jnp.float32),
        mesh=mesh,
        scratch_types=(
            [pltpu.VMEM((CHUNK_ELEMS,), jnp.float32) for _ in range(2)]
            + [pltpu.VMEM((CHUNK_ELEMS,), jnp.float32) for _ in range(NTOK_BUF)]
            + [pltpu.SemaphoreType.DMA for _ in range(2 + 4 * NTOK_BUF)]
        ),
    )


def _to_physical_tok(x):
    return (x.transpose(0, 2, 1)
             .reshape(BATCH, EMBED // 8, 8, TOKENS // 128, 128)
             .transpose(0, 1, 3, 2, 4)
             .reshape(BATCH * FLAT))


def _from_physical_tok(x):
    return (x.reshape(BATCH, EMBED // 8, TOKENS // 128, 8, 128)
             .transpose(0, 1, 3, 2, 4)
             .reshape(BATCH, EMBED, TOKENS)
             .transpose(0, 2, 1))


def _to_physical_pos(x):
    return (x.transpose(1, 0)
             .reshape(EMBED // 8, 8, TOKENS // 128, 128)
             .transpose(0, 2, 1, 3)
             .reshape(FLAT))


def kernel(encoded_tokens, position_embedding):
    tok_lin = _to_physical_tok(encoded_tokens)
    pos_lin = _to_physical_pos(position_embedding)
    out_lin = _posenc_sc()(tok_lin, pos_lin)
    return _from_physical_tok(out_lin)


# speedup vs baseline: 1.0330x; 1.0143x over previous
import functools

import jax
import jax.numpy as jnp
from jax import lax
from jax.experimental import pallas as pl
from jax.experimental.pallas import tpu as pltpu
from jax.experimental.pallas import tpu_sc as plsc

EMBED = 32
TOKENS = 131072
BATCH = 8
FLAT = TOKENS * EMBED

NC = 2
NS = 16
NW = NC * NS
ELEMS_PER_W = FLAT // NW
CHUNK_ELEMS = 16384
HALF = CHUNK_ELEMS // 2
NCHUNK = ELEMS_PER_W // CHUNK_ELEMS
NITER = NCHUNK * BATCH
NTOK_BUF = 4
PREF = 2


def _body(tok_hbm, pos_hbm, out_hbm, *refs):
    pos_v = refs[0:2]
    tok_v = refs[2:2 + NTOK_BUF]
    sem_pos = refs[2 + NTOK_BUF:4 + NTOK_BUF]
    sem_in = refs[4 + NTOK_BUF:4 + 2 * NTOK_BUF]
    sem_out = refs[4 + 2 * NTOK_BUF:4 + 3 * NTOK_BUF]

    wid = lax.axis_index("s") * NC + lax.axis_index("c")
    wbase = wid * ELEMS_PER_W

    def pos_in(j):
        return [pltpu.async_copy(
            pos_hbm.at[pl.ds(wbase + j * CHUNK_ELEMS, CHUNK_ELEMS)],
            pos_v[j % 2], sem_pos[j % 2])]

    def tok_in(it):
        j, b = divmod(it, BATCH)
        off = b * FLAT + wbase + j * CHUNK_ELEMS
        p = it % NTOK_BUF
        return [pltpu.async_copy(
            tok_hbm.at[pl.ds(off, CHUNK_ELEMS)], tok_v[p], sem_in[p])]

    def tok_out(it):
        j, b = divmod(it, BATCH)
        off = b * FLAT + wbase + j * CHUNK_ELEMS
        p = it % NTOK_BUF
        return [pltpu.async_copy(
            tok_v[p], out_hbm.at[pl.ds(off, CHUNK_ELEMS)], sem_out[p])]

    def wait(descs):
        for d in descs:
            d.wait()

    pending_pos = pos_in(0)
    pending_in = {i: tok_in(i) for i in range(PREF)}
    pending_out = {}

    for it in range(NITER):
        j, b = divmod(it, BATCH)
        if b == 0:
            wait(pending_pos)
            if j + 1 < NCHUNK:
                pending_pos = pos_in(j + 1)
        nx = it + PREF
        if nx < NITER:
            if nx - NTOK_BUF in pending_out:
                wait(pending_out.pop(nx - NTOK_BUF))
            pending_in[nx] = tok_in(nx)
        wait(pending_in.pop(it))

        tok_buf = tok_v[it % NTOK_BUF]
        pos_buf = pos_v[j % 2]

        @plsc.parallel_loop(0, CHUNK_ELEMS, step=16, unroll=8)
        def _(i):
            plsc.addupdate(tok_buf.at[pl.ds(i, 16)], pos_buf[pl.ds(i, 16)])

        pending_out[it] = tok_out(it)

    for it in sorted(pending_out):
        wait(pending_out[it])


@functools.cache
def _posenc_sc():
    mesh = plsc.VectorSubcoreMesh(core_axis_name="c", subcore_axis_name="s")
    return pl.kernel(
        _body,
        out_type=jax.ShapeDtypeStruct((BATCH * FLAT,), jnp.float32),
        mesh=mesh,
        scratch_types=(
            [pltpu.VMEM((CHUNK_ELEMS,), jnp.float32) for _ in range(2)]
            + [pltpu.VMEM((CHUNK_ELEMS,), jnp.float32) for _ in range(NTOK_BUF)]
            + [pltpu.SemaphoreType.DMA for _ in range(2 + 2 * NTOK_BUF)]
        ),
    )


def _to_physical_tok(x):
    return (x.transpose(0, 2, 1)
             .reshape(BATCH, EMBED // 8, 8, TOKENS // 128, 128)
             .transpose(0, 1, 3, 2, 4)
             .reshape(BATCH * FLAT))


def _from_physical_tok(x):
    return (x.reshape(BATCH, EMBED // 8, TOKENS // 128, 8, 128)
             .transpose(0, 1, 3, 2, 4)
             .reshape(BATCH, EMBED, TOKENS)
             .transpose(0, 2, 1))


def _to_physical_pos(x):
    return (x.transpose(1, 0)
             .reshape(EMBED // 8, 8, TOKENS // 128, 128)
             .transpose(0, 2, 1, 3)
             .reshape(FLAT))


def kernel(encoded_tokens, position_embedding):
    tok_lin = _to_physical_tok(encoded_tokens)
    pos_lin = _to_physical_pos(position_embedding)
    out_lin = _posenc_sc()(tok_lin, pos_lin)
    return _from_physical_tok(out_lin)


# speedup vs baseline: 1.0336x; 1.0006x over previous
import functools

import jax
import jax.numpy as jnp
from jax import lax
from jax.experimental import pallas as pl
from jax.experimental.pallas import tpu as pltpu
from jax.experimental.pallas import tpu_sc as plsc

EMBED = 32
TOKENS = 131072
BATCH = 8
FLAT = TOKENS * EMBED

NC = 2
NS = 16
NW = NC * NS
ELEMS_PER_W = FLAT // NW
CHUNK_ELEMS = 16384
HALF = CHUNK_ELEMS // 2
NCHUNK = ELEMS_PER_W // CHUNK_ELEMS
NITER = NCHUNK * BATCH
NTOK_BUF = 4
PREF = 2


def _body(tok_hbm, pos_hbm, out_hbm, *refs):
    pos_v = refs[0:2]
    tok_v = refs[2:2 + NTOK_BUF]
    sem_pos = refs[2 + NTOK_BUF:4 + NTOK_BUF]
    sem_in = refs[4 + NTOK_BUF:4 + 2 * NTOK_BUF]
    sem_out = refs[4 + 2 * NTOK_BUF:4 + 3 * NTOK_BUF]

    wid = lax.axis_index("s") * NC + lax.axis_index("c")
    wbase = wid * ELEMS_PER_W

    def pos_in(j):
        return [pltpu.async_copy(
            pos_hbm.at[pl.ds(wbase + j * CHUNK_ELEMS, CHUNK_ELEMS)],
            pos_v[j % 2], sem_pos[j % 2])]

    def tok_in(it):
        j, b = divmod(it, BATCH)
        off = b * FLAT + wbase + j * CHUNK_ELEMS
        p = it % NTOK_BUF
        return [pltpu.async_copy(
            tok_hbm.at[pl.ds(off, CHUNK_ELEMS)], tok_v[p], sem_in[p])]

    def tok_out(it):
        j, b = divmod(it, BATCH)
        off = b * FLAT + wbase + j * CHUNK_ELEMS
        p = it % NTOK_BUF
        return [pltpu.async_copy(
            tok_v[p], out_hbm.at[pl.ds(off, CHUNK_ELEMS)], sem_out[p])]

    def wait(descs):
        for d in descs:
            d.wait()

    pending_pos = pos_in(0)
    pending_in = {i: tok_in(i) for i in range(PREF)}
    pending_out = {}

    for it in range(NITER):
        j, b = divmod(it, BATCH)
        if b == 0:
            wait(pending_pos)
            if j + 1 < NCHUNK:
                pending_pos = pos_in(j + 1)
        nx = it + PREF
        if nx < NITER:
            if nx - NTOK_BUF in pending_out:
                wait(pending_out.pop(nx - NTOK_BUF))
            pending_in[nx] = tok_in(nx)
        wait(pending_in.pop(it))

        tok_buf = tok_v[it % NTOK_BUF]
        pos_buf = pos_v[j % 2]

        @plsc.parallel_loop(0, CHUNK_ELEMS, step=16, unroll=8)
        def _(i):
            plsc.addupdate(tok_buf.at[pl.ds(i, 16)], pos_buf[pl.ds(i, 16)])

        pending_out[it] = tok_out(it)

    for it in sorted(pending_out):
        wait(pending_out[it])


@functools.cache
def _posenc_sc():
    mesh = plsc.VectorSubcoreMesh(core_axis_name="c", subcore_axis_name="s")
    return pl.kernel(
        _body,
        out_type=jax.ShapeDtypeStruct((BATCH * FLAT,), jnp.float32),
        mesh=mesh,
        scratch_types=(
            [pltpu.VMEM((CHUNK_ELEMS,), jnp.float32) for _ in range(2)]
            + [pltpu.VMEM((CHUNK_ELEMS,), jnp.float32) for _ in range(NTOK_BUF)]
            + [pltpu.SemaphoreType.DMA for _ in range(2 + 2 * NTOK_BUF)]
        ),
        compiler_params=pltpu.CompilerParams(
            disable_bounds_checks=True, disable_semaphore_checks=True),
    )


def _to_physical_tok(x):
    return (x.transpose(0, 2, 1)
             .reshape(BATCH, EMBED // 8, 8, TOKENS // 128, 128)
             .transpose(0, 1, 3, 2, 4)
             .reshape(BATCH * FLAT))


def _from_physical_tok(x):
    return (x.reshape(BATCH, EMBED // 8, TOKENS // 128, 8, 128)
             .transpose(0, 1, 3, 2, 4)
             .reshape(BATCH, EMBED, TOKENS)
             .transpose(0, 2, 1))


def _to_physical_pos(x):
    return (x.transpose(1, 0)
             .reshape(EMBED // 8, 8, TOKENS // 128, 128)
             .transpose(0, 2, 1, 3)
             .reshape(FLAT))


def kernel(encoded_tokens, position_embedding):
    tok_lin = _to_physical_tok(encoded_tokens)
    pos_lin = _to_physical_pos(position_embedding)
    out_lin = _posenc_sc()(tok_lin, pos_lin)
    return _from_physical_tok(out_lin)
